# Initial kernel scaffold; baseline (speedup 1.0000x reference)
import jax
import jax.numpy as jnp
from jax.experimental import pallas as pl


def kernel(scores_map):
    raise NotImplementedError("write your pallas kernel here")


# speedup vs baseline: 1.0008x; 1.0008x over previous
import functools

import jax
import jax.numpy as jnp
import numpy as np
from jax.experimental import pallas as pl

RADIUS = 2
TOP_K = 4096
TEMP = 0.1
H = 512
W = 512
B = 8


def _mp_rows(x, h):
    z = jnp.zeros((2, x.shape[1]), x.dtype)
    xp = jnp.concatenate([z, x, z], axis=0)
    return functools.reduce(jnp.maximum, [xp[i:i + h] for i in range(5)])


def _mp_cols(x, w):
    z = jnp.zeros((x.shape[0], 2), x.dtype)
    xp = jnp.concatenate([z, x, z], axis=1)
    return functools.reduce(jnp.maximum, [xp[:, i:i + w] for i in range(5)])


def _maxpool5(x):
    return _mp_cols(_mp_rows(x, x.shape[0]), x.shape[1])


def _nms_body(x_ref, o_ref):
    x = x_ref[0]
    mp = _maxpool5(x)
    max_mask = x == mp
    for _ in range(2):
        supp = _maxpool5(max_mask.astype(jnp.float32)) > 0.0
        supp_scores = jnp.where(supp, 0.0, x)
        new_max = supp_scores == _maxpool5(supp_scores)
        max_mask = max_mask | (new_max & (~supp))
    nms = jnp.where(max_mask, x, 0.0)
    ri = jax.lax.broadcasted_iota(jnp.int32, (H, W), 0)
    ci = jax.lax.broadcasted_iota(jnp.int32, (H, W), 1)
    border = (ri < RADIUS) | (ri >= H - RADIUS) | (ci < RADIUS) | (ci >= W - RADIUS)
    o_ref[0] = jnp.where(border, 0.0, nms)


def _nms_pallas(scores):
    return pl.pallas_call(
        _nms_body,
        out_shape=jax.ShapeDtypeStruct((B, H, W), jnp.float32),
        grid=(B,),
        in_specs=[pl.BlockSpec((1, H, W), lambda b: (b, 0, 0))],
        out_specs=pl.BlockSpec((1, H, W), lambda b: (b, 0, 0)),
    )(scores)


def _hw_grid_np():
    ks = 2 * RADIUS + 1
    x = np.linspace(-RADIUS, RADIUS, ks)
    gi, gj = np.meshgrid(x, x, indexing='ij')
    return jnp.asarray(np.stack([gi, gj]).reshape(2, -1).T[:, [1, 0]], dtype=jnp.float32)


def _bilinear_img(img, xs, ys):
    h, w = img.shape
    x0 = jnp.floor(xs); y0 = jnp.floor(ys)
    x1 = x0 + 1.0; y1 = y0 + 1.0
    wa = (x1 - xs) * (y1 - ys)
    wb = (x1 - xs) * (ys - y0)
    wc = (xs - x0) * (y1 - ys)
    wd = (xs - x0) * (ys - y0)
    def g(yi, xi):
        yi = jnp.clip(yi.astype(jnp.int32), 0, h - 1)
        xi = jnp.clip(xi.astype(jnp.int32), 0, w - 1)
        return img[yi, xi]
    return wa * g(y0, x0) + wb * g(y1, x0) + wc * g(y0, x1) + wd * g(y1, x1)


def kernel(scores_map):
    b, c, h, w = scores_map.shape
    r = RADIUS
    ks = 2 * r + 1
    scores = scores_map[:, 0]
    nms = _nms_pallas(scores)
    flat = nms.reshape(b, -1)
    _, idx = jax.lax.top_k(flat, TOP_K)
    ys = idx // w
    xs = idx % w
    padded = jnp.pad(scores, ((0, 0), (r, r), (r, r)))
    def get_patch(img, y, x):
        return jax.lax.dynamic_slice(img, (y, x), (ks, ks)).reshape(-1)
    patches = jax.vmap(lambda img, yv, xv: jax.vmap(get_patch, (None, 0, 0))(img, yv, xv))(padded, ys, xs)
    hw = _hw_grid_np()
    max_v = jnp.max(patches, axis=-1, keepdims=True)
    x_exp = jnp.exp((patches - max_v) / TEMP)
    s = jnp.sum(x_exp, axis=-1)
    xy_res = jnp.einsum('bkp,pd->bkd', x_exp, hw) / s[..., None]
    dist2 = jnp.sum(((hw[None, None, :, :] - xy_res[:, :, None, :]) / r) ** 2, axis=-1)
    disp = jnp.sum(x_exp * dist2, axis=-1) / s
    nms_xy = jnp.stack([xs, ys], axis=-1).astype(jnp.float32)
    wh = jnp.array([w - 1, h - 1], dtype=jnp.float32)
    kpts = (nms_xy + xy_res) / wh * 2.0 - 1.0
    px = (kpts[..., 0] + 1.0) / 2.0 * (w - 1)
    py = (kpts[..., 1] + 1.0) / 2.0 * (h - 1)
    kptscores = jax.vmap(_bilinear_img)(scores, px, py)
    return kpts, disp, kptscores
